# baseline (speedup 1.0000x reference)
import functools

import jax
import jax.numpy as jnp
from jax import lax
from jax.experimental import pallas as pl
from jax.experimental.pallas import tpu as pltpu
from jax.experimental.pallas import tpu_sc as plsc

N_USER = 25000
N_ITEM = 25000
N_ALL = N_USER + N_ITEM
EMB = 64
HALF = 32
B = 4096

SUPER = 384
GB = 128
N_TILES = 16
NPAD_MAIN = 50176
NPAD_SIDE = 51200
SIDE_IBASE = 25600


def _chunks(tpr, maxrows):
    for n in range(1, 200):
        c = tpr // n
        if tpr % n == 0 and c <= maxrows and c % 8 == 0:
            return c, n
    raise ValueError(tpr)


def _pad_edges(e):
    q = N_TILES * SUPER
    return ((e + q - 1) // q) * q


def _spmm_body(n_pad, e_pad,
               col2_hbm, row_hbm, val_hbm, x_hbm, out_hbm,
               colb, rowb, valb, rows, acc, semg, sems, semi):
    NB = SUPER // GB
    cid = lax.axis_index("c")
    sid = lax.axis_index("s")
    tpr = n_pad // N_TILES
    z0 = sid * tpr
    t_edges = e_pad // N_TILES
    n_super = t_edges // SUPER
    t0 = sid * n_super

    zch, nz = _chunks(tpr, SUPER)
    def _zb(i, _):
        rows[0, i, pl.ds(0, 16)] = jnp.zeros((16,), jnp.float32)
        rows[0, i, pl.ds(16, 16)] = jnp.zeros((16,), jnp.float32)
        return 0
    lax.fori_loop(0, zch, _zb, 0)
    for k in range(nz):
        pltpu.sync_copy(rows.at[0, pl.ds(0, zch)],
                        acc.at[pl.ds(z0 + k * zch, zch)])
    plsc.subcore_barrier()

    def _fire_gathers(bi, gs):
        for j in range(NB):
            pltpu.async_copy(x_hbm.at[colb.at[bi, j]],
                             rows.at[bi, pl.ds(j * GB, GB)], semg)

    pltpu.sync_copy(col2_hbm.at[cid, t0], colb.at[0])
    pltpu.sync_copy(val_hbm.at[t0], valb.at[0])
    pltpu.sync_copy(row_hbm.at[t0], rowb.at[0])
    _fire_gathers(0, t0)

    def _super(s, _):
        b = lax.rem(s, 2)
        nb = 1 - b
        gs = t0 + s
        for j in range(NB):
            pltpu.make_async_copy(x_hbm.at[colb.at[b, j]],
                                  rows.at[b, pl.ds(j * GB, GB)], semg).wait()
        @pl.when(s + 1 < n_super)
        def _():
            pltpu.async_copy(col2_hbm.at[cid, gs + 1], colb.at[nb], semi)
            pltpu.async_copy(val_hbm.at[gs + 1], valb.at[nb], semi)

        @plsc.parallel_loop(0, SUPER // 16, unroll=4)
        def _scale(g):
            val16 = valb[b, pl.ds(g * 16, 16)]
            for l in range(16):
                v = val16[l]
                e = g * 16 + l
                rows[b, e, pl.ds(0, 16)] = rows[b, e, pl.ds(0, 16)] * v
                rows[b, e, pl.ds(16, 16)] = rows[b, e, pl.ds(16, 16)] * v

        @pl.when(s > 0)
        def _():
            for j in range(NB):
                pltpu.make_async_copy(rows.at[nb, pl.ds(j * GB, GB)],
                                      acc.at[rowb.at[nb, j]], sems).wait()
        @pl.when(s + 1 < n_super)
        def _():
            pltpu.async_copy(row_hbm.at[gs + 1], rowb.at[nb], semi)
        for j in range(NB):
            pltpu.async_copy(rows.at[b, pl.ds(j * GB, GB)],
                             acc.at[rowb.at[b, j]], sems, add=True)
        @pl.when(s + 1 < n_super)
        def _():
            pltpu.make_async_copy(col2_hbm.at[cid, gs + 1], colb.at[nb],
                                  semi).wait()
            pltpu.make_async_copy(val_hbm.at[gs + 1], valb.at[nb],
                                  semi).wait()
            pltpu.make_async_copy(row_hbm.at[gs + 1], rowb.at[nb],
                                  semi).wait()
            _fire_gathers(nb, gs + 1)
        return 0

    lax.fori_loop(0, n_super, _super, 0)
    bl = (n_super - 1) % 2
    for j in range(NB):
        pltpu.make_async_copy(rows.at[bl, pl.ds(j * GB, GB)],
                              acc.at[rowb.at[bl, j]], sems).wait()
    plsc.subcore_barrier()

    wch, nw = _chunks(tpr, SUPER)
    for k in range(nw):
        pltpu.sync_copy(acc.at[pl.ds(z0 + k * wch, wch)],
                        rows.at[0, pl.ds(0, wch)])
        pltpu.sync_copy(rows.at[0, pl.ds(0, wch)],
                        out_hbm.at[cid, pl.ds(z0 + k * wch, wch)])


def _spmm_launch(n_pad, e_pad, col2, row2, val2, x2):
    mesh = plsc.VectorSubcoreMesh(core_axis_name="c", subcore_axis_name="s")
    kfn = pl.kernel(
        functools.partial(_spmm_body, n_pad, e_pad),
        out_type=jax.ShapeDtypeStruct((2, n_pad, HALF), jnp.float32),
        mesh=mesh,
        scratch_types=[
            pltpu.VMEM((2, SUPER // GB, GB), jnp.int32),
            pltpu.VMEM((2, SUPER // GB, GB), jnp.int32),
            pltpu.VMEM((2, SUPER), jnp.float32),
            pltpu.VMEM((2, SUPER, HALF), jnp.float32),
            pltpu.VMEM_SHARED((n_pad, HALF), jnp.float32),
            pltpu.SemaphoreType.DMA,
            pltpu.SemaphoreType.DMA,
            pltpu.SemaphoreType.DMA,
        ],
        compiler_params=pltpu.CompilerParams(use_tc_tiling_on_sc=False),
    )
    return kfn(col2, row2, val2, x2)


def _prep_edges(row, col, val, row_base, col_lo, col_hi, pad_lo, pad_n):
    e_pad = _pad_edges(row.shape[0])
    pad = e_pad - row.shape[0]
    rowp = jnp.pad(row, (0, pad)) + row_base
    colp = jnp.pad(col, (0, pad))
    valp = jnp.pad(val, (0, pad))
    if pad:
        rowp = rowp.at[-pad:].set(pad_lo + jnp.arange(pad) % pad_n)
    col2 = jnp.stack([colp + col_lo, colp + col_hi])
    return e_pad, col2, rowp, valp


def _pack_edges(e_pad, col2, rowp, valp):
    ns = e_pad // SUPER
    return (e_pad,
            col2.reshape(2, ns, SUPER // GB, GB),
            rowp.reshape(ns, SUPER // GB, GB),
            valp.reshape(ns, SUPER))


def _prep_main(row, col, val):
    e_pad, col2, rowp, valp = _prep_edges(row, col, val, 0, 0, N_ALL,
                                          N_ALL, NPAD_MAIN - N_ALL)
    return _pack_edges(e_pad, col2, rowp, valp)


def _prep_side(u_row, u_col, u_val, i_row, i_col, i_val):
    eu, cu, ru, vu = _prep_edges(u_row, u_col, u_val, 0, 0, 2 * N_USER,
                                 N_USER, SIDE_IBASE - N_USER)
    ei, ci, ri, vi = _prep_edges(i_row, i_col, i_val, SIDE_IBASE,
                                 N_USER, 3 * N_USER,
                                 SIDE_IBASE + N_USER,
                                 NPAD_SIDE - SIDE_IBASE - N_USER)
    return _pack_edges(eu + ei,
                       jnp.concatenate([cu, ci], axis=1),
                       jnp.concatenate([ru, ri]),
                       jnp.concatenate([vu, vi]))


def _spmm_main(packed, x2):
    e_pad, col2, row2, val2 = packed
    return _spmm_launch(NPAD_MAIN, e_pad, col2, row2, val2, x2)


def _spmm_side_pair(packed, x_cat):
    e_pad, col2, row2, val2 = packed
    return _spmm_launch(NPAD_SIDE, e_pad, col2, row2, val2, x_cat)


def _lookup_body(t0, t1, t2, t3, t4, users, pos, neg, out_u, out_p, out_n,
                 idxb, rowsb, semg, semw):
    w = B // 32
    cid = lax.axis_index("c")
    sid = lax.axis_index("s")
    base = (sid * 2 + cid) * w
    tabs = (t0, t1, t2, t3, t4)
    idxs = (users, pos, neg)
    outs = (out_u, out_p, out_n)
    for o in range(3):
        pltpu.sync_copy(idxs[o].at[pl.ds(base, w)], idxb.at[o])
        if o > 0:
            for l in range(w // 16):
                idxb[o, pl.ds(l * 16, 16)] = (
                    idxb[o, pl.ds(l * 16, 16)] + N_USER)
        for k in range(5):
            pltpu.async_copy(tabs[k].at[idxb.at[o]],
                             rowsb.at[o * 5 + k], semg)
    for o in range(3):
        for k in range(5):
            pltpu.make_async_copy(tabs[k].at[idxb.at[o]],
                                  rowsb.at[o * 5 + k], semg).wait()
            pltpu.async_copy(rowsb.at[o * 5 + k],
                             outs[o].at[pl.ds(base, w),
                                        pl.ds(k * EMB, EMB)], semw)
    for o in range(3):
        for k in range(5):
            pltpu.make_async_copy(rowsb.at[o * 5 + k],
                                  outs[o].at[pl.ds(base, w),
                                             pl.ds(k * EMB, EMB)],
                                  semw).wait()


def _lookup_sc(tables, users, pos, neg):
    mesh = plsc.VectorSubcoreMesh(core_axis_name="c", subcore_axis_name="s")
    w = B // 32
    d = 5 * EMB
    out = jax.ShapeDtypeStruct((B, d), jnp.float32)
    kfn = pl.kernel(
        _lookup_body,
        out_type=(out, out, out),
        mesh=mesh,
        scratch_types=[
            pltpu.VMEM((3, w), jnp.int32),
            pltpu.VMEM((15, w, EMB), jnp.float32),
            pltpu.SemaphoreType.DMA,
            pltpu.SemaphoreType.DMA,
        ],
        compiler_params=pltpu.CompilerParams(use_tc_tiling_on_sc=False),
    )
    return kfn(*tables, users, pos, neg)


_RB = 1000


_SB = 200


def _mm_pair_body(x_ref, w_ref, o_ref):
    t = jnp.dot(x_ref[...], w_ref[0],
                preferred_element_type=jnp.float32)
    o_ref[0] = t[:, :HALF]
    o_ref[1] = t[:, HALF:]


def _tc_mm_split_pair(x, w_pair):
    n = x.shape[0]
    return pl.pallas_call(
        _mm_pair_body,
        grid=(n // _SB,),
        in_specs=[pl.BlockSpec((_SB, EMB), lambda i: (i, 0)),
                  pl.BlockSpec((1, EMB, EMB), lambda i: (i // 125, 0, 0))],
        out_specs=pl.BlockSpec((2, _SB, HALF), lambda i: (0, i, 0)),
        out_shape=jax.ShapeDtypeStruct((2, n, HALF), jnp.float32),
    )(x, w_pair)


def _side_blk(j):
    return jnp.where(j < 125, j, (SIDE_IBASE // _SB) + (j - 125))


def _elu_mm_pair_body(s_ref, b_ref, w_ref, o_ref):
    s = jnp.concatenate([s_ref[0], s_ref[1]], axis=1) + b_ref[0]
    h = jnp.where(s > 0, s, jnp.exp(s) - 1.0)
    t = jnp.dot(h, w_ref[0], preferred_element_type=jnp.float32)
    o_ref[0] = t[:, :HALF]
    o_ref[1] = t[:, HALF:]


def _tc_elu_mm_pair(s2, b0_pair, w1_pair):
    return pl.pallas_call(
        _elu_mm_pair_body,
        grid=(250,),
        in_specs=[pl.BlockSpec((2, _SB, HALF),
                               lambda i: (0, _side_blk(i), 0)),
                  pl.BlockSpec((1, 1, EMB), lambda i: (i // 125, 0, 0)),
                  pl.BlockSpec((1, EMB, EMB), lambda i: (i // 125, 0, 0))],
        out_specs=pl.BlockSpec((2, _SB, HALF), lambda i: (0, i, 0)),
        out_shape=jax.ShapeDtypeStruct((2, 2 * N_USER, HALF), jnp.float32),
    )(s2, b0_pair.reshape(2, 1, EMB), w1_pair)


def _relu_bias_pair_body(s_ref, b_ref, o_ref):
    s = jnp.concatenate([s_ref[0], s_ref[1]], axis=1) + b_ref[0]
    o_ref[...] = jnp.maximum(s, 0.0)


def _tc_relu_bias_pair(s2, b1_pair):
    return pl.pallas_call(
        _relu_bias_pair_body,
        grid=(250,),
        in_specs=[pl.BlockSpec((2, _SB, HALF),
                               lambda i: (0, _side_blk(i), 0)),
                  pl.BlockSpec((1, 1, EMB), lambda i: (i // 125, 0, 0))],
        out_specs=pl.BlockSpec((_SB, EMB), lambda i: (i, 0)),
        out_shape=jax.ShapeDtypeStruct((2 * N_USER, EMB), jnp.float32),
    )(s2, b1_pair.reshape(2, 1, EMB))


def _layer_body(s_ref, e_ref, wg_ref, bg_ref, wb_ref, bb_ref,
                eo_ref, no_ref):
    side = jnp.concatenate([s_ref[0], s_ref[1]], axis=1)
    ego = jnp.concatenate([e_ref[0], e_ref[1]], axis=1)
    sum_e = jnp.dot(side, wg_ref[...], preferred_element_type=jnp.float32)
    bi = jnp.dot(ego * side, wb_ref[...], preferred_element_type=jnp.float32)
    y = sum_e + bg_ref[...] + bi + bb_ref[...]
    z = jnp.where(y >= 0, y, 0.2 * y)
    eo_ref[0] = z[:, :HALF]
    eo_ref[1] = z[:, HALF:]
    nrm = jnp.sqrt(jnp.sum(z * z, axis=1, keepdims=True))
    no_ref[...] = z / jnp.maximum(nrm, 1e-12)


def _tc_layer(side, ego, wg, bg, wb, bb):
    n = N_ALL
    return pl.pallas_call(
        _layer_body,
        grid=(n // _RB,),
        in_specs=[pl.BlockSpec((2, _RB, HALF), lambda i: (0, i, 0)),
                  pl.BlockSpec((2, _RB, HALF), lambda i: (0, i, 0)),
                  pl.BlockSpec((EMB, EMB), lambda i: (0, 0)),
                  pl.BlockSpec((1, EMB), lambda i: (0, 0)),
                  pl.BlockSpec((EMB, EMB), lambda i: (0, 0)),
                  pl.BlockSpec((1, EMB), lambda i: (0, 0))],
        out_specs=(pl.BlockSpec((2, _RB, HALF), lambda i: (0, i, 0)),
                   pl.BlockSpec((_RB, EMB), lambda i: (i, 0))),
        out_shape=(jax.ShapeDtypeStruct((2, n, HALF), jnp.float32),
                   jax.ShapeDtypeStruct((n, EMB), jnp.float32)),
    )(side, ego, wg, bg, wb, bb)


def _side_gcn_pair(ego0, u_row, u_col, u_val, i_row, i_col, i_val,
                   u_W0, u_b0, u_W1, u_b1, i_W0, i_b0, i_W1, i_b1):
    w0p = jnp.stack([u_W0, i_W0])
    w1p = jnp.stack([u_W1, i_W1])
    b0p = jnp.stack([u_b0, i_b0])
    b1p = jnp.stack([u_b1, i_b1])
    packed = _prep_side(u_row, u_col, u_val, i_row, i_col, i_val)
    t = _tc_mm_split_pair(ego0, w0p)
    s1 = _spmm_side_pair(packed, t.reshape(4 * N_USER, HALF))
    h = _tc_elu_mm_pair(s1, b0p, w1p)
    s2 = _spmm_side_pair(packed, h.reshape(4 * N_USER, HALF))
    return _tc_relu_bias_pair(s2, b1p)


def kernel(users, pos_items, neg_items, adj_row, adj_col, adj_val,
           u_row, u_col, u_val, i_row, i_col, i_val,
           user_emb, item_emb,
           W_gc_0, b_gc_0, W_bi_0, b_bi_0,
           W_gc_1, b_gc_1, W_bi_1, b_bi_1,
           W_gc_2, b_gc_2, W_bi_2, b_bi_2,
           u_W0, u_b0, u_W1, u_b1, i_W0, i_b0, i_W1, i_b1):
    ego0 = jnp.concatenate([user_emb, item_emb], axis=0)
    lo = jnp.concatenate([user_emb[:, :HALF], item_emb[:, :HALF]], axis=0)
    hi = jnp.concatenate([user_emb[:, HALF:], item_emb[:, HALF:]], axis=0)
    ego_split = jnp.stack([lo, hi])

    uie = _side_gcn_pair(ego0, u_row, u_col, u_val, i_row, i_col, i_val,
                         u_W0, u_b0, u_W1, u_b1, i_W0, i_b0, i_W1, i_b1)

    wg = (W_gc_0, W_gc_1, W_gc_2)
    bg = (b_gc_0, b_gc_1, b_gc_2)
    wb = (W_bi_0, W_bi_1, W_bi_2)
    bb = (b_bi_0, b_bi_1, b_bi_2)
    normed = []
    adj_packed = _prep_main(adj_row, adj_col, adj_val)
    for k in range(3):
        side = _spmm_main(adj_packed,
                          ego_split.reshape(2 * N_ALL, HALF))
        ego_split, nk = _tc_layer(side, ego_split, wg[k], bg[k], wb[k], bb[k])
        normed.append(nk)

    tables = (ego0, normed[0], normed[1], normed[2], uie)
    return _lookup_sc(tables, users, pos_items, neg_items)


# speedup vs baseline: 1.1205x; 1.1205x over previous
import functools

import jax
import jax.numpy as jnp
from jax import lax
from jax.experimental import pallas as pl
from jax.experimental.pallas import tpu as pltpu
from jax.experimental.pallas import tpu_sc as plsc

N_USER = 25000
N_ITEM = 25000
N_ALL = N_USER + N_ITEM
EMB = 64
HALF = 32
B = 4096

SUPER = 384
GB = 128
N_TILES = 16
NPAD_MAIN = 50176
NPAD_SIDE = 51200
SIDE_IBASE = 25600


def _chunks(tpr, maxrows):
    for n in range(1, 200):
        c = tpr // n
        if tpr % n == 0 and c <= maxrows and c % 8 == 0:
            return c, n
    raise ValueError(tpr)


def _pad_edges(e):
    q = N_TILES * SUPER
    return ((e + q - 1) // q) * q


def _spmm_body(n_pad, e_pad,
               col2_hbm, row_hbm, val_hbm, x_hbm, out_hbm,
               colb, rowb, valb, rows, acc, semg, sems, semi):
    NB = SUPER // GB
    cid = lax.axis_index("c")
    sid = lax.axis_index("s")
    tpr = n_pad // N_TILES
    z0 = sid * tpr
    t_edges = e_pad // N_TILES
    n_super = t_edges // SUPER
    t0 = sid * n_super

    zch, nz = _chunks(tpr, SUPER)
    def _zb(i, _):
        rows[0, i, pl.ds(0, 16)] = jnp.zeros((16,), jnp.float32)
        rows[0, i, pl.ds(16, 16)] = jnp.zeros((16,), jnp.float32)
        return 0
    lax.fori_loop(0, zch, _zb, 0)
    for k in range(nz):
        pltpu.sync_copy(rows.at[0, pl.ds(0, zch)],
                        acc.at[pl.ds(z0 + k * zch, zch)])
    plsc.subcore_barrier()

    def _fire_gathers(bi, gs):
        for j in range(NB):
            pltpu.async_copy(x_hbm.at[colb.at[bi, j]],
                             rows.at[bi, pl.ds(j * GB, GB)], semg)

    pltpu.sync_copy(col2_hbm.at[cid, t0], colb.at[0])
    pltpu.sync_copy(val_hbm.at[t0], valb.at[0])
    pltpu.sync_copy(row_hbm.at[t0], rowb.at[0])
    _fire_gathers(0, t0)

    def _super(s, _):
        b = lax.rem(s, 2)
        nb = 1 - b
        gs = t0 + s
        for j in range(NB):
            pltpu.make_async_copy(x_hbm.at[colb.at[b, j]],
                                  rows.at[b, pl.ds(j * GB, GB)], semg).wait()
        @pl.when(s + 1 < n_super)
        def _():
            pltpu.async_copy(col2_hbm.at[cid, gs + 1], colb.at[nb], semi)
            pltpu.async_copy(val_hbm.at[gs + 1], valb.at[nb], semi)

        @pl.when(s > 0)
        def _():
            for j in range(NB):
                pltpu.make_async_copy(rows.at[nb, pl.ds(j * GB, GB)],
                                      acc.at[rowb.at[nb, j]], sems).wait()
        @pl.when(s + 1 < n_super)
        def _():
            pltpu.async_copy(row_hbm.at[gs + 1], rowb.at[nb], semi)
            pltpu.make_async_copy(col2_hbm.at[cid, gs + 1], colb.at[nb],
                                  semi).wait()
            pltpu.make_async_copy(val_hbm.at[gs + 1], valb.at[nb],
                                  semi).wait()
            _fire_gathers(nb, gs + 1)

        @plsc.parallel_loop(0, SUPER // 16, unroll=2)
        def _scale(g):
            val16 = valb[b, pl.ds(g * 16, 16)]
            for l in range(16):
                v = val16[l]
                e = g * 16 + l
                rows[b, e, pl.ds(0, 16)] = rows[b, e, pl.ds(0, 16)] * v
                rows[b, e, pl.ds(16, 16)] = rows[b, e, pl.ds(16, 16)] * v

        for j in range(NB):
            pltpu.async_copy(rows.at[b, pl.ds(j * GB, GB)],
                             acc.at[rowb.at[b, j]], sems, add=True)
        @pl.when(s + 1 < n_super)
        def _():
            pltpu.make_async_copy(row_hbm.at[gs + 1], rowb.at[nb],
                                  semi).wait()
        return 0

    lax.fori_loop(0, n_super, _super, 0)
    bl = (n_super - 1) % 2
    for j in range(NB):
        pltpu.make_async_copy(rows.at[bl, pl.ds(j * GB, GB)],
                              acc.at[rowb.at[bl, j]], sems).wait()
    plsc.subcore_barrier()

    wch, nw = _chunks(tpr, SUPER)
    for k in range(nw):
        pltpu.sync_copy(acc.at[pl.ds(z0 + k * wch, wch)],
                        rows.at[0, pl.ds(0, wch)])
        pltpu.sync_copy(rows.at[0, pl.ds(0, wch)],
                        out_hbm.at[cid, pl.ds(z0 + k * wch, wch)])


def _spmm_launch(n_pad, e_pad, col2, row2, val2, x2):
    mesh = plsc.VectorSubcoreMesh(core_axis_name="c", subcore_axis_name="s")
    kfn = pl.kernel(
        functools.partial(_spmm_body, n_pad, e_pad),
        out_type=jax.ShapeDtypeStruct((2, n_pad, HALF), jnp.float32),
        mesh=mesh,
        scratch_types=[
            pltpu.VMEM((2, SUPER // GB, GB), jnp.int32),
            pltpu.VMEM((2, SUPER // GB, GB), jnp.int32),
            pltpu.VMEM((2, SUPER), jnp.float32),
            pltpu.VMEM((2, SUPER, HALF), jnp.float32),
            pltpu.VMEM_SHARED((n_pad, HALF), jnp.float32),
            pltpu.SemaphoreType.DMA,
            pltpu.SemaphoreType.DMA,
            pltpu.SemaphoreType.DMA,
        ],
        compiler_params=pltpu.CompilerParams(use_tc_tiling_on_sc=False),
    )
    return kfn(col2, row2, val2, x2)


def _prep_edges(row, col, val, row_base, col_lo, col_hi, pad_lo, pad_n):
    e_pad = _pad_edges(row.shape[0])
    pad = e_pad - row.shape[0]
    rowp = jnp.pad(row, (0, pad)) + row_base
    colp = jnp.pad(col, (0, pad))
    valp = jnp.pad(val, (0, pad))
    if pad:
        rowp = rowp.at[-pad:].set(pad_lo + jnp.arange(pad) % pad_n)
    col2 = jnp.stack([colp + col_lo, colp + col_hi])
    return e_pad, col2, rowp, valp


def _pack_edges(e_pad, col2, rowp, valp):
    ns = e_pad // SUPER
    return (e_pad,
            col2.reshape(2, ns, SUPER // GB, GB),
            rowp.reshape(ns, SUPER // GB, GB),
            valp.reshape(ns, SUPER))


def _prep_main(row, col, val):
    e_pad, col2, rowp, valp = _prep_edges(row, col, val, 0, 0, N_ALL,
                                          N_ALL, NPAD_MAIN - N_ALL)
    return _pack_edges(e_pad, col2, rowp, valp)


def _prep_side(u_row, u_col, u_val, i_row, i_col, i_val):
    eu, cu, ru, vu = _prep_edges(u_row, u_col, u_val, 0, 0, 2 * N_USER,
                                 N_USER, SIDE_IBASE - N_USER)
    ei, ci, ri, vi = _prep_edges(i_row, i_col, i_val, SIDE_IBASE,
                                 N_USER, 3 * N_USER,
                                 SIDE_IBASE + N_USER,
                                 NPAD_SIDE - SIDE_IBASE - N_USER)
    return _pack_edges(eu + ei,
                       jnp.concatenate([cu, ci], axis=1),
                       jnp.concatenate([ru, ri]),
                       jnp.concatenate([vu, vi]))


def _spmm_main(packed, x2):
    e_pad, col2, row2, val2 = packed
    return _spmm_launch(NPAD_MAIN, e_pad, col2, row2, val2, x2)


def _spmm_side_pair(packed, x_cat):
    e_pad, col2, row2, val2 = packed
    return _spmm_launch(NPAD_SIDE, e_pad, col2, row2, val2, x_cat)


def _lookup_body(t0, t1, t2, t3, t4, users, pos, neg, out_u, out_p, out_n,
                 idxb, rowsb, semg, semw):
    w = B // 32
    cid = lax.axis_index("c")
    sid = lax.axis_index("s")
    base = (sid * 2 + cid) * w
    tabs = (t0, t1, t2, t3, t4)
    idxs = (users, pos, neg)
    outs = (out_u, out_p, out_n)
    for o in range(3):
        pltpu.sync_copy(idxs[o].at[pl.ds(base, w)], idxb.at[o])
        if o > 0:
            for l in range(w // 16):
                idxb[o, pl.ds(l * 16, 16)] = (
                    idxb[o, pl.ds(l * 16, 16)] + N_USER)
        for k in range(5):
            pltpu.async_copy(tabs[k].at[idxb.at[o]],
                             rowsb.at[o * 5 + k], semg)
    for o in range(3):
        for k in range(5):
            pltpu.make_async_copy(tabs[k].at[idxb.at[o]],
                                  rowsb.at[o * 5 + k], semg).wait()
            pltpu.async_copy(rowsb.at[o * 5 + k],
                             outs[o].at[pl.ds(base, w),
                                        pl.ds(k * EMB, EMB)], semw)
    for o in range(3):
        for k in range(5):
            pltpu.make_async_copy(rowsb.at[o * 5 + k],
                                  outs[o].at[pl.ds(base, w),
                                             pl.ds(k * EMB, EMB)],
                                  semw).wait()


def _lookup_sc(tables, users, pos, neg):
    mesh = plsc.VectorSubcoreMesh(core_axis_name="c", subcore_axis_name="s")
    w = B // 32
    d = 5 * EMB
    out = jax.ShapeDtypeStruct((B, d), jnp.float32)
    kfn = pl.kernel(
        _lookup_body,
        out_type=(out, out, out),
        mesh=mesh,
        scratch_types=[
            pltpu.VMEM((3, w), jnp.int32),
            pltpu.VMEM((15, w, EMB), jnp.float32),
            pltpu.SemaphoreType.DMA,
            pltpu.SemaphoreType.DMA,
        ],
        compiler_params=pltpu.CompilerParams(use_tc_tiling_on_sc=False),
    )
    return kfn(*tables, users, pos, neg)


_RB = 1000


_SB = 200


def _mm_pair_body(x_ref, w_ref, o_ref):
    t = jnp.dot(x_ref[...], w_ref[0],
                preferred_element_type=jnp.float32)
    o_ref[0] = t[:, :HALF]
    o_ref[1] = t[:, HALF:]


def _tc_mm_split_pair(x, w_pair):
    n = x.shape[0]
    return pl.pallas_call(
        _mm_pair_body,
        grid=(n // _SB,),
        in_specs=[pl.BlockSpec((_SB, EMB), lambda i: (i, 0)),
                  pl.BlockSpec((1, EMB, EMB), lambda i: (i // 125, 0, 0))],
        out_specs=pl.BlockSpec((2, _SB, HALF), lambda i: (0, i, 0)),
        out_shape=jax.ShapeDtypeStruct((2, n, HALF), jnp.float32),
    )(x, w_pair)


def _side_blk(j):
    return jnp.where(j < 125, j, (SIDE_IBASE // _SB) + (j - 125))


def _elu_mm_pair_body(s_ref, b_ref, w_ref, o_ref):
    s = jnp.concatenate([s_ref[0], s_ref[1]], axis=1) + b_ref[0]
    h = jnp.where(s > 0, s, jnp.exp(s) - 1.0)
    t = jnp.dot(h, w_ref[0], preferred_element_type=jnp.float32)
    o_ref[0] = t[:, :HALF]
    o_ref[1] = t[:, HALF:]


def _tc_elu_mm_pair(s2, b0_pair, w1_pair):
    return pl.pallas_call(
        _elu_mm_pair_body,
        grid=(250,),
        in_specs=[pl.BlockSpec((2, _SB, HALF),
                               lambda i: (0, _side_blk(i), 0)),
                  pl.BlockSpec((1, 1, EMB), lambda i: (i // 125, 0, 0)),
                  pl.BlockSpec((1, EMB, EMB), lambda i: (i // 125, 0, 0))],
        out_specs=pl.BlockSpec((2, _SB, HALF), lambda i: (0, i, 0)),
        out_shape=jax.ShapeDtypeStruct((2, 2 * N_USER, HALF), jnp.float32),
    )(s2, b0_pair.reshape(2, 1, EMB), w1_pair)


def _relu_bias_pair_body(s_ref, b_ref, o_ref):
    s = jnp.concatenate([s_ref[0], s_ref[1]], axis=1) + b_ref[0]
    o_ref[...] = jnp.maximum(s, 0.0)


def _tc_relu_bias_pair(s2, b1_pair):
    return pl.pallas_call(
        _relu_bias_pair_body,
        grid=(250,),
        in_specs=[pl.BlockSpec((2, _SB, HALF),
                               lambda i: (0, _side_blk(i), 0)),
                  pl.BlockSpec((1, 1, EMB), lambda i: (i // 125, 0, 0))],
        out_specs=pl.BlockSpec((_SB, EMB), lambda i: (i, 0)),
        out_shape=jax.ShapeDtypeStruct((2 * N_USER, EMB), jnp.float32),
    )(s2, b1_pair.reshape(2, 1, EMB))


def _layer_body(s_ref, e_ref, wg_ref, bg_ref, wb_ref, bb_ref,
                eo_ref, no_ref):
    side = jnp.concatenate([s_ref[0], s_ref[1]], axis=1)
    ego = jnp.concatenate([e_ref[0], e_ref[1]], axis=1)
    sum_e = jnp.dot(side, wg_ref[...], preferred_element_type=jnp.float32)
    bi = jnp.dot(ego * side, wb_ref[...], preferred_element_type=jnp.float32)
    y = sum_e + bg_ref[...] + bi + bb_ref[...]
    z = jnp.where(y >= 0, y, 0.2 * y)
    eo_ref[0] = z[:, :HALF]
    eo_ref[1] = z[:, HALF:]
    nrm = jnp.sqrt(jnp.sum(z * z, axis=1, keepdims=True))
    no_ref[...] = z / jnp.maximum(nrm, 1e-12)


def _tc_layer(side, ego, wg, bg, wb, bb):
    n = N_ALL
    return pl.pallas_call(
        _layer_body,
        grid=(n // _RB,),
        in_specs=[pl.BlockSpec((2, _RB, HALF), lambda i: (0, i, 0)),
                  pl.BlockSpec((2, _RB, HALF), lambda i: (0, i, 0)),
                  pl.BlockSpec((EMB, EMB), lambda i: (0, 0)),
                  pl.BlockSpec((1, EMB), lambda i: (0, 0)),
                  pl.BlockSpec((EMB, EMB), lambda i: (0, 0)),
                  pl.BlockSpec((1, EMB), lambda i: (0, 0))],
        out_specs=(pl.BlockSpec((2, _RB, HALF), lambda i: (0, i, 0)),
                   pl.BlockSpec((_RB, EMB), lambda i: (i, 0))),
        out_shape=(jax.ShapeDtypeStruct((2, n, HALF), jnp.float32),
                   jax.ShapeDtypeStruct((n, EMB), jnp.float32)),
    )(side, ego, wg, bg, wb, bb)


def _side_gcn_pair(ego0, u_row, u_col, u_val, i_row, i_col, i_val,
                   u_W0, u_b0, u_W1, u_b1, i_W0, i_b0, i_W1, i_b1):
    w0p = jnp.stack([u_W0, i_W0])
    w1p = jnp.stack([u_W1, i_W1])
    b0p = jnp.stack([u_b0, i_b0])
    b1p = jnp.stack([u_b1, i_b1])
    packed = _prep_side(u_row, u_col, u_val, i_row, i_col, i_val)
    t = _tc_mm_split_pair(ego0, w0p)
    s1 = _spmm_side_pair(packed, t.reshape(4 * N_USER, HALF))
    h = _tc_elu_mm_pair(s1, b0p, w1p)
    s2 = _spmm_side_pair(packed, h.reshape(4 * N_USER, HALF))
    return _tc_relu_bias_pair(s2, b1p)


def kernel(users, pos_items, neg_items, adj_row, adj_col, adj_val,
           u_row, u_col, u_val, i_row, i_col, i_val,
           user_emb, item_emb,
           W_gc_0, b_gc_0, W_bi_0, b_bi_0,
           W_gc_1, b_gc_1, W_bi_1, b_bi_1,
           W_gc_2, b_gc_2, W_bi_2, b_bi_2,
           u_W0, u_b0, u_W1, u_b1, i_W0, i_b0, i_W1, i_b1):
    ego0 = jnp.concatenate([user_emb, item_emb], axis=0)
    lo = jnp.concatenate([user_emb[:, :HALF], item_emb[:, :HALF]], axis=0)
    hi = jnp.concatenate([user_emb[:, HALF:], item_emb[:, HALF:]], axis=0)
    ego_split = jnp.stack([lo, hi])

    uie = _side_gcn_pair(ego0, u_row, u_col, u_val, i_row, i_col, i_val,
                         u_W0, u_b0, u_W1, u_b1, i_W0, i_b0, i_W1, i_b1)

    wg = (W_gc_0, W_gc_1, W_gc_2)
    bg = (b_gc_0, b_gc_1, b_gc_2)
    wb = (W_bi_0, W_bi_1, W_bi_2)
    bb = (b_bi_0, b_bi_1, b_bi_2)
    normed = []
    adj_packed = _prep_main(adj_row, adj_col, adj_val)
    for k in range(3):
        side = _spmm_main(adj_packed,
                          ego_split.reshape(2 * N_ALL, HALF))
        ego_split, nk = _tc_layer(side, ego_split, wg[k], bg[k], wb[k], bb[k])
        normed.append(nk)

    tables = (ego0, normed[0], normed[1], normed[2], uie)
    return _lookup_sc(tables, users, pos_items, neg_items)


# speedup vs baseline: 1.1296x; 1.0081x over previous
import functools

import jax
import jax.numpy as jnp
from jax import lax
from jax.experimental import pallas as pl
from jax.experimental.pallas import tpu as pltpu
from jax.experimental.pallas import tpu_sc as plsc

N_USER = 25000
N_ITEM = 25000
N_ALL = N_USER + N_ITEM
EMB = 64
HALF = 32
B = 4096

SUPER = 384
GB = 128
N_TILES = 16
NPAD_MAIN = 50176
NPAD_SIDE = 51200
SIDE_IBASE = 25600


def _chunks(tpr, maxrows):
    for n in range(1, 200):
        c = tpr // n
        if tpr % n == 0 and c <= maxrows and c % 8 == 0:
            return c, n
    raise ValueError(tpr)


def _pad_edges(e):
    q = N_TILES * SUPER
    return ((e + q - 1) // q) * q


def _spmm_body(n_pad, e_pad,
               col2_hbm, row_hbm, val_hbm, x_hbm, out_hbm,
               colb, rowb, valb, rows, acc, semg, sems, semi):
    NB = SUPER // GB
    cid = lax.axis_index("c")
    sid = lax.axis_index("s")
    tpr = n_pad // N_TILES
    z0 = sid * tpr
    t_edges = e_pad // N_TILES
    n_super = t_edges // SUPER
    t0 = sid * n_super

    zch, nz = _chunks(tpr, SUPER)
    def _zb(i, _):
        rows[0, i, pl.ds(0, 16)] = jnp.zeros((16,), jnp.float32)
        rows[0, i, pl.ds(16, 16)] = jnp.zeros((16,), jnp.float32)
        return 0
    lax.fori_loop(0, zch, _zb, 0)
    for k in range(nz):
        pltpu.async_copy(rows.at[0, pl.ds(0, zch)],
                         acc.at[pl.ds(z0 + k * zch, zch)], semi)
    for k in range(nz):
        pltpu.make_async_copy(rows.at[0, pl.ds(0, zch)],
                              acc.at[pl.ds(z0 + k * zch, zch)], semi).wait()
    plsc.subcore_barrier()

    def _fire_gathers(bi, gs):
        for j in range(NB):
            pltpu.async_copy(x_hbm.at[colb.at[bi, j]],
                             rows.at[bi, pl.ds(j * GB, GB)], semg)

    pltpu.sync_copy(col2_hbm.at[cid, t0], colb.at[0])
    pltpu.sync_copy(val_hbm.at[t0], valb.at[0])
    pltpu.sync_copy(row_hbm.at[t0], rowb.at[0])
    _fire_gathers(0, t0)

    def _super(s, _):
        b = lax.rem(s, 2)
        nb = 1 - b
        gs = t0 + s
        for j in range(NB):
            pltpu.make_async_copy(x_hbm.at[colb.at[b, j]],
                                  rows.at[b, pl.ds(j * GB, GB)], semg).wait()
        @pl.when(s + 1 < n_super)
        def _():
            pltpu.async_copy(col2_hbm.at[cid, gs + 1], colb.at[nb], semi)
            pltpu.async_copy(val_hbm.at[gs + 1], valb.at[nb], semi)

        @pl.when(s > 0)
        def _():
            for j in range(NB):
                pltpu.make_async_copy(rows.at[nb, pl.ds(j * GB, GB)],
                                      acc.at[rowb.at[nb, j]], sems).wait()
        @pl.when(s + 1 < n_super)
        def _():
            pltpu.async_copy(row_hbm.at[gs + 1], rowb.at[nb], semi)
            pltpu.make_async_copy(col2_hbm.at[cid, gs + 1], colb.at[nb],
                                  semi).wait()
            pltpu.make_async_copy(val_hbm.at[gs + 1], valb.at[nb],
                                  semi).wait()
            _fire_gathers(nb, gs + 1)

        @plsc.parallel_loop(0, SUPER // 16, unroll=2)
        def _scale(g):
            val16 = valb[b, pl.ds(g * 16, 16)]
            for l in range(16):
                v = val16[l]
                e = g * 16 + l
                rows[b, e, pl.ds(0, 16)] = rows[b, e, pl.ds(0, 16)] * v
                rows[b, e, pl.ds(16, 16)] = rows[b, e, pl.ds(16, 16)] * v

        for j in range(NB):
            pltpu.async_copy(rows.at[b, pl.ds(j * GB, GB)],
                             acc.at[rowb.at[b, j]], sems, add=True)
        @pl.when(s + 1 < n_super)
        def _():
            pltpu.make_async_copy(row_hbm.at[gs + 1], rowb.at[nb],
                                  semi).wait()
        return 0

    lax.fori_loop(0, n_super, _super, 0)
    bl = (n_super - 1) % 2
    for j in range(NB):
        pltpu.make_async_copy(rows.at[bl, pl.ds(j * GB, GB)],
                              acc.at[rowb.at[bl, j]], sems).wait()
    plsc.subcore_barrier()

    wch, nw = _chunks(tpr, SUPER)
    def _wb_out(k):
        return pltpu.make_async_copy(
            rows.at[k % 2, pl.ds(0, wch)],
            out_hbm.at[cid, pl.ds(z0 + k * wch, wch)], sems)
    for k in range(nw):
        if k >= 2:
            _wb_out(k - 2).wait()
        pltpu.sync_copy(acc.at[pl.ds(z0 + k * wch, wch)],
                        rows.at[k % 2, pl.ds(0, wch)])
        pltpu.async_copy(rows.at[k % 2, pl.ds(0, wch)],
                         out_hbm.at[cid, pl.ds(z0 + k * wch, wch)], sems)
    for k in range(max(0, nw - 2), nw):
        _wb_out(k).wait()


def _spmm_launch(n_pad, e_pad, col2, row2, val2, x2):
    mesh = plsc.VectorSubcoreMesh(core_axis_name="c", subcore_axis_name="s")
    kfn = pl.kernel(
        functools.partial(_spmm_body, n_pad, e_pad),
        out_type=jax.ShapeDtypeStruct((2, n_pad, HALF), jnp.float32),
        mesh=mesh,
        scratch_types=[
            pltpu.VMEM((2, SUPER // GB, GB), jnp.int32),
            pltpu.VMEM((2, SUPER // GB, GB), jnp.int32),
            pltpu.VMEM((2, SUPER), jnp.float32),
            pltpu.VMEM((2, SUPER, HALF), jnp.float32),
            pltpu.VMEM_SHARED((n_pad, HALF), jnp.float32),
            pltpu.SemaphoreType.DMA,
            pltpu.SemaphoreType.DMA,
            pltpu.SemaphoreType.DMA,
        ],
        compiler_params=pltpu.CompilerParams(use_tc_tiling_on_sc=False),
    )
    return kfn(col2, row2, val2, x2)


def _prep_edges(row, col, val, row_base, col_lo, col_hi, pad_lo, pad_n):
    e_pad = _pad_edges(row.shape[0])
    pad = e_pad - row.shape[0]
    rowp = jnp.pad(row, (0, pad)) + row_base
    colp = jnp.pad(col, (0, pad))
    valp = jnp.pad(val, (0, pad))
    if pad:
        rowp = rowp.at[-pad:].set(pad_lo + jnp.arange(pad) % pad_n)
    col2 = jnp.stack([colp + col_lo, colp + col_hi])
    return e_pad, col2, rowp, valp


def _pack_edges(e_pad, col2, rowp, valp):
    ns = e_pad // SUPER
    return (e_pad,
            col2.reshape(2, ns, SUPER // GB, GB),
            rowp.reshape(ns, SUPER // GB, GB),
            valp.reshape(ns, SUPER))


def _prep_main(row, col, val):
    e_pad, col2, rowp, valp = _prep_edges(row, col, val, 0, 0, N_ALL,
                                          N_ALL, NPAD_MAIN - N_ALL)
    return _pack_edges(e_pad, col2, rowp, valp)


def _prep_side(u_row, u_col, u_val, i_row, i_col, i_val):
    eu, cu, ru, vu = _prep_edges(u_row, u_col, u_val, 0, 0, 2 * N_USER,
                                 N_USER, SIDE_IBASE - N_USER)
    ei, ci, ri, vi = _prep_edges(i_row, i_col, i_val, SIDE_IBASE,
                                 N_USER, 3 * N_USER,
                                 SIDE_IBASE + N_USER,
                                 NPAD_SIDE - SIDE_IBASE - N_USER)
    return _pack_edges(eu + ei,
                       jnp.concatenate([cu, ci], axis=1),
                       jnp.concatenate([ru, ri]),
                       jnp.concatenate([vu, vi]))


def _spmm_main(packed, x2):
    e_pad, col2, row2, val2 = packed
    return _spmm_launch(NPAD_MAIN, e_pad, col2, row2, val2, x2)


def _spmm_side_pair(packed, x_cat):
    e_pad, col2, row2, val2 = packed
    return _spmm_launch(NPAD_SIDE, e_pad, col2, row2, val2, x_cat)


def _lookup_body(t0, t1, t2, t3, t4, users, pos, neg, out_u, out_p, out_n,
                 idxb, rowsb, semg, semw):
    w = B // 32
    cid = lax.axis_index("c")
    sid = lax.axis_index("s")
    base = (sid * 2 + cid) * w
    tabs = (t0, t1, t2, t3, t4)
    idxs = (users, pos, neg)
    outs = (out_u, out_p, out_n)
    for o in range(3):
        pltpu.sync_copy(idxs[o].at[pl.ds(base, w)], idxb.at[o])
        if o > 0:
            for l in range(w // 16):
                idxb[o, pl.ds(l * 16, 16)] = (
                    idxb[o, pl.ds(l * 16, 16)] + N_USER)
        for k in range(5):
            pltpu.async_copy(tabs[k].at[idxb.at[o]],
                             rowsb.at[o * 5 + k], semg)
    for o in range(3):
        for k in range(5):
            pltpu.make_async_copy(tabs[k].at[idxb.at[o]],
                                  rowsb.at[o * 5 + k], semg).wait()
            pltpu.async_copy(rowsb.at[o * 5 + k],
                             outs[o].at[pl.ds(base, w),
                                        pl.ds(k * EMB, EMB)], semw)
    for o in range(3):
        for k in range(5):
            pltpu.make_async_copy(rowsb.at[o * 5 + k],
                                  outs[o].at[pl.ds(base, w),
                                             pl.ds(k * EMB, EMB)],
                                  semw).wait()


def _lookup_sc(tables, users, pos, neg):
    mesh = plsc.VectorSubcoreMesh(core_axis_name="c", subcore_axis_name="s")
    w = B // 32
    d = 5 * EMB
    out = jax.ShapeDtypeStruct((B, d), jnp.float32)
    kfn = pl.kernel(
        _lookup_body,
        out_type=(out, out, out),
        mesh=mesh,
        scratch_types=[
            pltpu.VMEM((3, w), jnp.int32),
            pltpu.VMEM((15, w, EMB), jnp.float32),
            pltpu.SemaphoreType.DMA,
            pltpu.SemaphoreType.DMA,
        ],
        compiler_params=pltpu.CompilerParams(use_tc_tiling_on_sc=False),
    )
    return kfn(*tables, users, pos, neg)


_RB = 1000


_SB = 200


def _mm_pair_body(x_ref, w_ref, o_ref):
    t = jnp.dot(x_ref[...], w_ref[0],
                preferred_element_type=jnp.float32)
    o_ref[0] = t[:, :HALF]
    o_ref[1] = t[:, HALF:]


def _tc_mm_split_pair(x, w_pair):
    n = x.shape[0]
    return pl.pallas_call(
        _mm_pair_body,
        grid=(n // _SB,),
        in_specs=[pl.BlockSpec((_SB, EMB), lambda i: (i, 0)),
                  pl.BlockSpec((1, EMB, EMB), lambda i: (i // 125, 0, 0))],
        out_specs=pl.BlockSpec((2, _SB, HALF), lambda i: (0, i, 0)),
        out_shape=jax.ShapeDtypeStruct((2, n, HALF), jnp.float32),
    )(x, w_pair)


def _side_blk(j):
    return jnp.where(j < 125, j, (SIDE_IBASE // _SB) + (j - 125))


def _elu_mm_pair_body(s_ref, b_ref, w_ref, o_ref):
    s = jnp.concatenate([s_ref[0], s_ref[1]], axis=1) + b_ref[0]
    h = jnp.where(s > 0, s, jnp.exp(s) - 1.0)
    t = jnp.dot(h, w_ref[0], preferred_element_type=jnp.float32)
    o_ref[0] = t[:, :HALF]
    o_ref[1] = t[:, HALF:]


def _tc_elu_mm_pair(s2, b0_pair, w1_pair):
    return pl.pallas_call(
        _elu_mm_pair_body,
        grid=(250,),
        in_specs=[pl.BlockSpec((2, _SB, HALF),
                               lambda i: (0, _side_blk(i), 0)),
                  pl.BlockSpec((1, 1, EMB), lambda i: (i // 125, 0, 0)),
                  pl.BlockSpec((1, EMB, EMB), lambda i: (i // 125, 0, 0))],
        out_specs=pl.BlockSpec((2, _SB, HALF), lambda i: (0, i, 0)),
        out_shape=jax.ShapeDtypeStruct((2, 2 * N_USER, HALF), jnp.float32),
    )(s2, b0_pair.reshape(2, 1, EMB), w1_pair)


def _relu_bias_pair_body(s_ref, b_ref, o_ref):
    s = jnp.concatenate([s_ref[0], s_ref[1]], axis=1) + b_ref[0]
    o_ref[...] = jnp.maximum(s, 0.0)


def _tc_relu_bias_pair(s2, b1_pair):
    return pl.pallas_call(
        _relu_bias_pair_body,
        grid=(250,),
        in_specs=[pl.BlockSpec((2, _SB, HALF),
                               lambda i: (0, _side_blk(i), 0)),
                  pl.BlockSpec((1, 1, EMB), lambda i: (i // 125, 0, 0))],
        out_specs=pl.BlockSpec((_SB, EMB), lambda i: (i, 0)),
        out_shape=jax.ShapeDtypeStruct((2 * N_USER, EMB), jnp.float32),
    )(s2, b1_pair.reshape(2, 1, EMB))


def _layer_body(s_ref, e_ref, wg_ref, bg_ref, wb_ref, bb_ref,
                eo_ref, no_ref):
    side = jnp.concatenate([s_ref[0], s_ref[1]], axis=1)
    ego = jnp.concatenate([e_ref[0], e_ref[1]], axis=1)
    sum_e = jnp.dot(side, wg_ref[...], preferred_element_type=jnp.float32)
    bi = jnp.dot(ego * side, wb_ref[...], preferred_element_type=jnp.float32)
    y = sum_e + bg_ref[...] + bi + bb_ref[...]
    z = jnp.where(y >= 0, y, 0.2 * y)
    eo_ref[0] = z[:, :HALF]
    eo_ref[1] = z[:, HALF:]
    nrm = jnp.sqrt(jnp.sum(z * z, axis=1, keepdims=True))
    no_ref[...] = z / jnp.maximum(nrm, 1e-12)


def _tc_layer(side, ego, wg, bg, wb, bb):
    n = N_ALL
    return pl.pallas_call(
        _layer_body,
        grid=(n // _RB,),
        in_specs=[pl.BlockSpec((2, _RB, HALF), lambda i: (0, i, 0)),
                  pl.BlockSpec((2, _RB, HALF), lambda i: (0, i, 0)),
                  pl.BlockSpec((EMB, EMB), lambda i: (0, 0)),
                  pl.BlockSpec((1, EMB), lambda i: (0, 0)),
                  pl.BlockSpec((EMB, EMB), lambda i: (0, 0)),
                  pl.BlockSpec((1, EMB), lambda i: (0, 0))],
        out_specs=(pl.BlockSpec((2, _RB, HALF), lambda i: (0, i, 0)),
                   pl.BlockSpec((_RB, EMB), lambda i: (i, 0))),
        out_shape=(jax.ShapeDtypeStruct((2, n, HALF), jnp.float32),
                   jax.ShapeDtypeStruct((n, EMB), jnp.float32)),
    )(side, ego, wg, bg, wb, bb)


def _side_gcn_pair(ego0, u_row, u_col, u_val, i_row, i_col, i_val,
                   u_W0, u_b0, u_W1, u_b1, i_W0, i_b0, i_W1, i_b1):
    w0p = jnp.stack([u_W0, i_W0])
    w1p = jnp.stack([u_W1, i_W1])
    b0p = jnp.stack([u_b0, i_b0])
    b1p = jnp.stack([u_b1, i_b1])
    packed = _prep_side(u_row, u_col, u_val, i_row, i_col, i_val)
    t = _tc_mm_split_pair(ego0, w0p)
    s1 = _spmm_side_pair(packed, t.reshape(4 * N_USER, HALF))
    h = _tc_elu_mm_pair(s1, b0p, w1p)
    s2 = _spmm_side_pair(packed, h.reshape(4 * N_USER, HALF))
    return _tc_relu_bias_pair(s2, b1p)


def kernel(users, pos_items, neg_items, adj_row, adj_col, adj_val,
           u_row, u_col, u_val, i_row, i_col, i_val,
           user_emb, item_emb,
           W_gc_0, b_gc_0, W_bi_0, b_bi_0,
           W_gc_1, b_gc_1, W_bi_1, b_bi_1,
           W_gc_2, b_gc_2, W_bi_2, b_bi_2,
           u_W0, u_b0, u_W1, u_b1, i_W0, i_b0, i_W1, i_b1):
    ego0 = jnp.concatenate([user_emb, item_emb], axis=0)
    lo = jnp.concatenate([user_emb[:, :HALF], item_emb[:, :HALF]], axis=0)
    hi = jnp.concatenate([user_emb[:, HALF:], item_emb[:, HALF:]], axis=0)
    ego_split = jnp.stack([lo, hi])

    uie = _side_gcn_pair(ego0, u_row, u_col, u_val, i_row, i_col, i_val,
                         u_W0, u_b0, u_W1, u_b1, i_W0, i_b0, i_W1, i_b1)

    wg = (W_gc_0, W_gc_1, W_gc_2)
    bg = (b_gc_0, b_gc_1, b_gc_2)
    wb = (W_bi_0, W_bi_1, W_bi_2)
    bb = (b_bi_0, b_bi_1, b_bi_2)
    normed = []
    adj_packed = _prep_main(adj_row, adj_col, adj_val)
    for k in range(3):
        side = _spmm_main(adj_packed,
                          ego_split.reshape(2 * N_ALL, HALF))
        ego_split, nk = _tc_layer(side, ego_split, wg[k], bg[k], wb[k], bb[k])
        normed.append(nk)

    tables = (ego0, normed[0], normed[1], normed[2], uie)
    return _lookup_sc(tables, users, pos_items, neg_items)
